# baseline (speedup 1.0000x reference)
import jax
import jax.numpy as jnp
import numpy as np
from jax import lax
from jax.experimental import pallas as pl
from jax.experimental.pallas import tpu as pltpu
from jax.experimental.pallas import tpu_sc as plsc

_ATTR_DIM = 64
_NFREQ = 3
_MU = 11.0
_N = 4096 * 200
_NWORKERS = 32
_ROWS_PER_W = _N // _NWORKERS
_C = 128
_NCHUNKS = _ROWS_PER_W // _C
_FWIDTH = 16


def _fourier_table() -> np.ndarray:
    b = np.arange(256)
    xi = ((b >> 4) & 15).astype(np.float32)
    yi = (b & 15).astype(np.float32)
    xn = xi / np.float32(_MU - 1.0) * np.float32(2.0) - np.float32(1.0)
    yn = yi / np.float32(_MU - 1.0) * np.float32(2.0) - np.float32(1.0)
    freqs = (2.0 ** np.arange(_NFREQ)).astype(np.float32)
    xs = xn[:, None] * freqs[None, :]
    ys = yn[:, None] * freqs[None, :]
    t = np.zeros((256, _FWIDTH), dtype=np.float32)
    t[:, 0:3] = np.cos(xs)
    t[:, 3:6] = np.sin(xs)
    t[:, 6:9] = np.cos(ys)
    t[:, 9:12] = np.sin(ys)
    return t


_TXY = _fourier_table()


def _sc_body(td_hbm, w_hbm, txy_hbm, out_hbm,
             td_v, idx_a, embed_v, four_p, txy_v, sem_a):
    wid = lax.axis_index("s") * 2 + lax.axis_index("c")
    lanes = lax.iota(jnp.int32, 16)
    pltpu.sync_copy(txy_hbm, txy_v)

    def chunk(t, _):
        base = wid * _ROWS_PER_W + t * _C
        pltpu.sync_copy(td_hbm.at[pl.ds(base * 3, 3 * _C)], td_v)

        def grp_idx(g, _):
            r3 = (g * 16 + lanes) * 3
            a = plsc.load_gather(td_v, [r3 + 1])
            idx_a[pl.ds(g * 16, 16)] = a & 255
            return 0

        lax.fori_loop(0, _C // 16, grp_idx, 0)
        cp_a = pltpu.async_copy(w_hbm.at[idx_a], embed_v, sem_a)

        def grp_four(g, _):
            r = g * 16 + lanes
            b = plsc.load_gather(td_v, [r * 3]) & 255
            for c in range(12):
                col = jnp.full((16,), c, jnp.int32)
                fc = plsc.load_gather(txy_v, [b, col])
                plsc.store_scatter(four_p, [r, col], fc)
            v = plsc.load_gather(td_v, [r * 3 + 2]).astype(jnp.float32)
            plsc.store_scatter(four_p, [r, jnp.full((16,), 12, jnp.int32)], v)
            return 0

        lax.fori_loop(0, _C // 16, grp_four, 0)
        cp_a.wait()
        pltpu.sync_copy(embed_v, out_hbm.at[pl.ds(base, _C), pl.ds(0, 64)])
        pltpu.sync_copy(four_p, out_hbm.at[pl.ds(base, _C), pl.ds(64, 13)])
        return 0

    lax.fori_loop(0, _NCHUNKS, chunk, 0)


@jax.jit
def _run(td_flat, w, txy):
    mesh = plsc.VectorSubcoreMesh(core_axis_name="c", subcore_axis_name="s")
    f = pl.kernel(
        _sc_body,
        out_type=jax.ShapeDtypeStruct((_N, 77), jnp.float32),
        mesh=mesh,
        scratch_types=[
            pltpu.VMEM((3 * _C,), jnp.int32),
            pltpu.VMEM((_C,), jnp.int32),
            pltpu.VMEM((_C, _ATTR_DIM), jnp.float32),
            pltpu.VMEM((_C, 13), jnp.float32),
            pltpu.VMEM((256, _FWIDTH), jnp.float32),
            pltpu.SemaphoreType.DMA,
        ],
        compiler_params=pltpu.CompilerParams(
            use_tc_tiling_on_sc=False, needs_layout_passes=False),
    )
    return f(td_flat, w, txy)


def kernel(td, W):
    td_flat = td.reshape(-1)
    out = _run(td_flat, W, jnp.asarray(_TXY))
    return out.reshape(td.shape[0], td.shape[1], 77)
